# baseline (speedup 1.0000x reference)
import functools

import jax
import jax.numpy as jnp
from jax import lax
from jax.experimental import pallas as pl
from jax.experimental.pallas import tpu as pltpu
from jax.experimental.pallas import tpu_sc as plsc

D_MODEL = 4096
NUM_EXPERTS = 64
TOP_K = 8
TOKENS = 16384
HOT_PENALTY = 0.01
COLD_BOOST = 0.02

BLOCK = 1024
NW = 32
TPW = TOKENS // NW


def _probs_kernel(h_ref, gwt_ref, loads_ref, p_ref):
    logits = jnp.dot(h_ref[...], gwt_ref[...],
                     preferred_element_type=jnp.float32)
    loads = loads_ref[...]
    target = TOP_K / NUM_EXPERTS
    adj = (jnp.where(loads > target * 1.5, -HOT_PENALTY, 0.0)
           + jnp.where(loads < target * 0.5, COLD_BOOST, 0.0))
    logits = logits + adj
    m = jnp.max(logits, axis=-1, keepdims=True)
    e = jnp.exp(logits - m)
    s = jnp.sum(e, axis=-1, keepdims=True)
    p_ref[...] = e / s


def _tc_probs(hidden_states, gwt, loads2d):
    return pl.pallas_call(
        _probs_kernel,
        grid=(TOKENS // BLOCK,),
        in_specs=[
            pl.BlockSpec((BLOCK, D_MODEL), lambda b: (b, 0)),
            pl.BlockSpec((D_MODEL, NUM_EXPERTS), lambda b: (0, 0)),
            pl.BlockSpec((1, NUM_EXPERTS), lambda b: (0, 0)),
        ],
        out_specs=pl.BlockSpec((BLOCK, NUM_EXPERTS), lambda b: (b, 0)),
        out_shape=jax.ShapeDtypeStruct((TOKENS, NUM_EXPERTS), jnp.float32),
        compiler_params=pltpu.CompilerParams(
            dimension_semantics=("arbitrary",),
        ),
    )(hidden_states, gwt, loads2d)


def _merge16(av, ai, bv, bi):
    rbv = lax.rev(bv, (0,))
    rbi = lax.rev(bi, (0,))
    take_a = av >= rbv
    return jnp.where(take_a, av, rbv), jnp.where(take_a, ai, rbi)


@functools.partial(
    pl.kernel,
    mesh=plsc.VectorSubcoreMesh(core_axis_name="c", subcore_axis_name="s"),
    out_type=(
        jax.ShapeDtypeStruct((TOKENS * TOP_K,), jnp.int32),
        jax.ShapeDtypeStruct((TOKENS * TOP_K,), jnp.float32),
    ),
    scratch_types=[
        pltpu.VMEM((TPW, NUM_EXPERTS), jnp.float32),
        pltpu.VMEM((TPW * TOP_K,), jnp.int32),
        pltpu.VMEM((TPW * TOP_K,), jnp.float32),
    ],
    compiler_params=pltpu.CompilerParams(needs_layout_passes=False),
)
def _sc_topk(probs_hbm, idx_hbm, w_hbm, pv, iv, wv):
    wid = lax.axis_index("s") * 2 + lax.axis_index("c")
    base = wid * TPW
    pltpu.sync_copy(probs_hbm.at[pl.ds(base, TPW)], pv)

    lane = lax.iota(jnp.int32, 16)
    topmask = lane < TOP_K

    def body(t, _):
        svs = []
        sis = []
        for q in range(4):
            x = pv[t, pl.ds(q * 16, 16)]
            sv, si = plsc.sort_key_val(x, lane + q * 16, descending=True)
            svs.append(sv)
            sis.append(si)
        m0v, m0i = _merge16(svs[0], sis[0], svs[1], sis[1])
        m1v, m1i = _merge16(svs[2], sis[2], svs[3], sis[3])
        s0v, s0i = plsc.sort_key_val(m0v, m0i, descending=True)
        s1v, s1i = plsc.sort_key_val(m1v, m1i, descending=True)
        fv, fi = _merge16(s0v, s0i, s1v, s1i)
        gv, gi = plsc.sort_key_val(fv, fi, descending=True)
        tot = jnp.sum(jnp.where(topmask, gv, 0.0))
        dest = t * TOP_K + lane
        plsc.store_scatter(iv, [dest], gi, mask=topmask)
        plsc.store_scatter(wv, [dest], gv / tot, mask=topmask)
        return _

    lax.fori_loop(0, TPW, body, None)
    pltpu.sync_copy(iv, idx_hbm.at[pl.ds(base * TOP_K, TPW * TOP_K)])
    pltpu.sync_copy(wv, w_hbm.at[pl.ds(base * TOP_K, TPW * TOP_K)])


def kernel(hidden_states, gate_weight, expert_loads):
    gwt = gate_weight.T
    loads2d = expert_loads.reshape(1, NUM_EXPERTS)
    probs = _tc_probs(hidden_states, gwt, loads2d)
    idx, w = _sc_topk(probs)
    return (idx.reshape(TOKENS, TOP_K), w.reshape(TOKENS, TOP_K))


# speedup vs baseline: 1.5351x; 1.5351x over previous
import jax
import jax.numpy as jnp
from jax.experimental import pallas as pl
from jax.experimental.pallas import tpu as pltpu

D_MODEL = 4096
NUM_EXPERTS = 64
TOP_K = 8
TOKENS = 16384
HOT_PENALTY = 0.01
COLD_BOOST = 0.02

BLOCK = 1024


def _router_kernel(h_ref, gwt_ref, loads_ref, idx_ref, w_ref):
    logits = jnp.dot(h_ref[...], gwt_ref[...],
                     preferred_element_type=jnp.float32)

    loads = loads_ref[...]
    target = TOP_K / NUM_EXPERTS
    adj = (jnp.where(loads > target * 1.5, -HOT_PENALTY, 0.0)
           + jnp.where(loads < target * 0.5, COLD_BOOST, 0.0))

    logits = logits + adj

    m = jnp.max(logits, axis=-1, keepdims=True)
    e = jnp.exp(logits - m)
    s = jnp.sum(e, axis=-1, keepdims=True)
    probs = (e / s).T

    row = jax.lax.broadcasted_iota(jnp.int32, (NUM_EXPERTS, BLOCK), 0)
    sub8 = jax.lax.broadcasted_iota(jnp.int32, (TOP_K, BLOCK), 0)
    cur = probs
    out_v = jnp.zeros((TOP_K, BLOCK), jnp.float32)
    out_i = jnp.zeros((TOP_K, BLOCK), jnp.int32)
    for j in range(TOP_K):
        mv = jnp.max(cur, axis=0, keepdims=True)
        am = jnp.min(jnp.where(cur == mv, row, NUM_EXPERTS), axis=0,
                     keepdims=True)
        out_v = jnp.where(sub8 == j, mv, out_v)
        out_i = jnp.where(sub8 == j, am, out_i)
        cur = jnp.where(row == am, -1.0, cur)

    w = out_v / jnp.sum(out_v, axis=0, keepdims=True)
    idx_ref[...] = out_i.T
    w_ref[...] = w.T


def kernel(hidden_states, gate_weight, expert_loads):
    gwt = gate_weight.T
    loads2d = expert_loads.reshape(1, NUM_EXPERTS)
    n_blocks = TOKENS // BLOCK
    grid = (n_blocks,)
    out_shapes = (
        jax.ShapeDtypeStruct((TOKENS, TOP_K), jnp.int32),
        jax.ShapeDtypeStruct((TOKENS, TOP_K), jnp.float32),
    )
    idx, w = pl.pallas_call(
        _router_kernel,
        grid=grid,
        in_specs=[
            pl.BlockSpec((BLOCK, D_MODEL), lambda b: (b, 0)),
            pl.BlockSpec((D_MODEL, NUM_EXPERTS), lambda b: (0, 0)),
            pl.BlockSpec((1, NUM_EXPERTS), lambda b: (0, 0)),
        ],
        out_specs=(
            pl.BlockSpec((BLOCK, TOP_K), lambda b: (b, 0)),
            pl.BlockSpec((BLOCK, TOP_K), lambda b: (b, 0)),
        ),
        out_shape=out_shapes,
        compiler_params=pltpu.CompilerParams(
            dimension_semantics=("arbitrary",),
        ),
    )(hidden_states, gwt, loads2d)
    return (idx, w)
